# Initial kernel scaffold; baseline (speedup 1.0000x reference)
import jax
import jax.numpy as jnp
from jax.experimental import pallas as pl


def kernel(atom_types, positions, edge_features, params, edge_index):
    raise NotImplementedError("write your pallas kernel here")


# speedup vs baseline: 8.8997x; 8.8997x over previous
import functools

import jax
import jax.numpy as jnp
from jax import lax
from jax.experimental import pallas as pl
from jax.experimental.pallas import tpu as pltpu

_S = 64
_VN = 16
_EF = 32
_R = 16
_DMAX = 20.0
_MN = 100.0
_NL = 3
_BLK = 1024
_PAD_UNIT = 4096

_SRC_W = {0: 80, 1: 128, 2: 160}
_DST_W = {0: 80, 1: 80, 2: 112}
_POS_OFF_SRC = {0: 64, 1: 115, 2: 115}


def _silu(x):
    return x * jax.nn.sigmoid(x)


def _ln(x, g, b):
    m = jnp.mean(x, -1, keepdims=True)
    v = jnp.mean((x - m) ** 2, -1, keepdims=True)
    return (x - m) / jnp.sqrt(v + 1e-5) * g + b


def _mlp(x, p, residual=None):
    h = _silu(x @ p['W1'] + p['b1'])
    h = _silu(h @ p['W2'] + p['b2'])
    if residual is not None:
        h = h + residual
    return _ln(h, p['g'], p['b'])


def _norm(x, axis=-1, keepdims=False):
    return jnp.sqrt(jnp.clip(jnp.sum(x * x, axis=axis, keepdims=keepdims), 1e-8, None))


def _gvp(s, V, p):
    Vh = jnp.einsum('nic,ih->nhc', V, p['Wh'])
    vn = _norm(Vh, axis=-1)
    so = _silu(jnp.concatenate([s, vn], -1) @ p['Ws'] + p['bs'])
    Vo = jnp.einsum('nhc,ho->noc', Vh, p['Wv'])
    gate = jax.nn.sigmoid(so @ p['Wg'] + p['bg'])
    return so, Vo * gate[..., None]


def _structured_mats(wh_last, Wv):
    I3 = jnp.eye(3, dtype=jnp.float32)
    M3 = (wh_last[None, :, None] * I3[:, None, :]).reshape(3, 51)
    K = jnp.repeat(jnp.eye(17, dtype=jnp.float32), 3, axis=0)
    WK = (Wv[:, None, :, None] * I3[None, :, None, :]).reshape(51, 48)
    RG = jnp.repeat(jnp.eye(16, dtype=jnp.float32), 3, axis=1)
    T3 = jnp.tile(I3, (1, 16))
    return M3, K, WK, RG, T3


def _msg_body(mode, *refs):
    if mode == 0:
        (ef, gs, gd, We1, be1, We2, be2, ge, be,
         Wz, bs, Wsv, wh2, Wg, bg, wvo, RG, T3, msg_o, e_o) = refs
    elif mode == 1:
        (e_r, gs, gd, Wz, bs, Wg, bg, M3, K, WK, RG, msg_o) = refs
    else:
        (e_r, gs, gd, Wz, bs, Wg, bg, M3, K, WK, RG,
         W1z, b1, W2, b2, lg, lb, msg_o) = refs

    gsv = gs[:]
    gdv = gd[:]
    A = gsv[:, 0:64]
    B = gdv[:, 0:64]
    po = _POS_OFF_SRC[mode]
    ps = gsv[:, po:po + 3]
    pd = gdv[:, 64:67]

    diff = ps - pd
    n2 = jnp.sum(diff * diff, axis=1, keepdims=True)
    dij = jnp.sqrt(jnp.clip(n2, 1e-8, None)) + 1e-8
    xd = diff / dij
    sigma = _DMAX / _R
    mu = lax.broadcasted_iota(jnp.int32, (1, _R), 1).astype(jnp.float32) * (
        _DMAX / (_R - 1))
    dmat = jnp.exp(-(((dij - mu) / sigma) ** 2))

    if mode == 0:
        h = _silu(ef[:] @ We1[:] + be1[:])
        h = _silu(h @ We2[:] + be2[:])
        ev = _ln(h, ge[:], be[:])
        e_o[:] = ev
    else:
        ev = e_r[:]

    if mode == 2:
        U1a = gsv[:, 118:150]
        U1b = gdv[:, 67:99]
        h1 = _silu(U1a + U1b + jnp.concatenate([ev, dmat], 1) @ W1z[:] + b1[:])
        h2 = _silu(h1 @ W2[:] + b2[:])
        ev = _ln(h2 + ev, lg[:], lb[:])

    if mode == 0:
        nx2 = jnp.sum(xd * xd, axis=1, keepdims=True)
        vn = jnp.sqrt(jnp.clip(nx2 * wh2[:], 1e-8, None))
        z = A + B + jnp.concatenate([ev, dmat], 1) @ Wz[:] + vn @ Wsv[:] + bs[:]
    else:
        Vh = gsv[:, 64:115] + xd @ M3[:]
        vn = jnp.sqrt(jnp.clip((Vh * Vh) @ K[:], 1e-8, None))
        z = A + B + jnp.concatenate([ev, dmat, vn], 1) @ Wz[:] + bs[:]

    so = _silu(z)
    gate = jax.nn.sigmoid(so @ Wg[:] + bg[:])
    if mode == 0:
        outV = ((gate * wvo[:]) @ RG[:]) * (xd @ T3[:])
    else:
        outV = (Vh @ WK[:]) * (gate @ RG[:])
    msg_o[:] = jnp.concatenate([so, outV], 1)


def _msg_layer(mode, Ep, data_arrays, weights):
    grid = Ep // _BLK
    in_specs = [pl.BlockSpec((_BLK, a.shape[1]), lambda i: (i, 0))
                for a in data_arrays]
    in_specs += [pl.BlockSpec(w.shape, lambda i: (0, 0)) for w in weights]
    out_shapes = [jax.ShapeDtypeStruct((Ep, 112), jnp.float32)]
    out_specs = [pl.BlockSpec((_BLK, 112), lambda i: (i, 0))]
    if mode == 0:
        out_shapes.append(jax.ShapeDtypeStruct((Ep, _EF), jnp.float32))
        out_specs.append(pl.BlockSpec((_BLK, _EF), lambda i: (i, 0)))
    fn = pl.pallas_call(
        functools.partial(_msg_body, mode),
        grid=(grid,),
        in_specs=in_specs,
        out_specs=out_specs,
        out_shape=out_shapes,
        compiler_params=pltpu.CompilerParams(
            dimension_semantics=("arbitrary",)),
    )
    return fn(*data_arrays, *weights)


def _msg_weights(mode, params, u_mlp=None):
    p = params['msg']
    Wh, Ws, bs = p['Wh'], p['Ws'], p['bs']
    Wv, Wg, bg = p['Wv'], p['Wg'], p['bg']
    wh_last = Wh[16]
    Ws_e = Ws[128:160]
    Ws_d = Ws[160:176]
    Ws_v = Ws[176:193]
    M3, K, WK, RG, T3 = _structured_mats(wh_last, Wv)
    r = lambda x: x.reshape(1, -1)
    if mode == 0:
        Wz = jnp.concatenate([Ws_e, Ws_d], 0)
        wvo = (wh_last @ Wv).reshape(1, 16)
        wh2 = (wh_last * wh_last).reshape(1, 17)
        return [Wz, r(bs), Ws_v, wh2, Wg, r(bg), wvo, RG, T3]
    Wz = jnp.concatenate([Ws_e, Ws_d, Ws_v], 0)
    w = [Wz, r(bs), Wg, r(bg), M3, K, WK, RG]
    if mode == 2:
        W1 = u_mlp['W1']
        W1z = jnp.concatenate([W1[128:160], W1[160:176]], 0)
        w += [W1z, r(u_mlp['b1']), u_mlp['W2'], r(u_mlp['b2']),
              r(u_mlp['g']), r(u_mlp['b'])]
    return w


def _gather(table, idx):
    return jnp.take(table, idx, axis=0)


def _pad_cols(x, w):
    return jnp.pad(x, ((0, 0), (0, w - x.shape[1])))


def kernel(atom_types, positions, edge_features, params, edge_index):
    N = atom_types.shape[0]
    E = edge_features.shape[0]
    Ep = ((E + _PAD_UNIT - 1) // _PAD_UNIT) * _PAD_UNIT
    npad = Ep - E

    src = edge_index[0]
    dst = edge_index[1]
    pad_ids = jnp.arange(npad, dtype=src.dtype)
    srcp = jnp.concatenate([src, pad_ids % N])
    dstp = jnp.concatenate([dst, N + (pad_ids % 64)])
    efp = jnp.pad(edge_features.astype(jnp.float32), ((0, npad), (0, 3)))

    s = _mlp(atom_types, params['scalar_emb'])
    V = jnp.zeros((N, _VN, 3), jnp.float32)
    pos = positions

    e = None
    for i in range(_NL):
        cp = params['convs'][i]
        mode = i
        msgp = cp['msg']
        Ws = msgp['Ws']
        A = s @ Ws[0:64]
        B = s @ Ws[64:128]
        cols_src = [A, pos]
        if mode != 0:
            C = jnp.einsum('nic,ih->nhc', V, msgp['Wh'][:16]).reshape(N, 51)
            cols_src = [A, C, pos]
        cols_dst = [B, pos]
        if mode == 2:
            W1 = params['edge_upd'][0]['W1']
            cols_src.append(s @ W1[0:64])
            cols_dst.append(s @ W1[64:128])
        Tsrc = _pad_cols(jnp.concatenate(cols_src, 1), _SRC_W[mode])
        Tdst = _pad_cols(jnp.concatenate(cols_dst, 1), _DST_W[mode])
        Tsrc = jnp.concatenate([Tsrc, jnp.zeros((64, _SRC_W[mode]))], 0)
        Tdst = jnp.concatenate([Tdst, jnp.zeros((64, _DST_W[mode]))], 0)

        Gsrc = _gather(Tsrc, srcp)
        Gdst = _gather(Tdst, dstp)

        weights = _msg_weights(mode, cp,
                               params['edge_upd'][0] if mode == 2 else None)
        if mode == 0:
            ee = params['edge_emb']
            We1 = jnp.concatenate([ee['W1'], jnp.zeros((3, _EF))], 0)
            r = lambda x: x.reshape(1, -1)
            emb_w = [We1, r(ee['b1']), ee['W2'], r(ee['b2']),
                     r(ee['g']), r(ee['b'])]
            msg, e = _msg_layer(0, Ep, [efp, Gsrc, Gdst], emb_w + weights)
        else:
            (msg,) = _msg_layer(mode, Ep, [e, Gsrc, Gdst], weights)

        agg = jax.ops.segment_sum(msg, dstp, num_segments=N + 64)[:N]
        s = s + agg[:, :64] / _MN
        V = V + agg[:, 64:].reshape(N, _VN, 3) / _MN

        ds, dV = _gvp(s, V, cp['upd'])
        s = _ln(s + ds, cp['ln_g'], cp['ln_b'])
        V = V + dV

        if i == 1:
            pu = params['pos_upd'][0]
            s1, v1 = _gvp(s, V, pu['g1'])
            s2, v2 = _gvp(s1, v1, pu['g2'])
            s3, v3 = _gvp(s2, v2, pu['g3'])
            pos = pos + v3[:, 0, :]

    pooled = jnp.mean(s, axis=0, keepdims=True)
    pr = params['pred']
    return _silu(pooled @ pr['W1'] + pr['b1']) @ pr['W2'] + pr['b2']


# speedup vs baseline: 14.9630x; 1.6813x over previous
import functools

import jax
import jax.numpy as jnp
from jax import lax
from jax.experimental import pallas as pl
from jax.experimental.pallas import tpu as pltpu
from jax.experimental.pallas import tpu_sc as plsc

_S = 64
_VN = 16
_EF = 32
_R = 16
_DMAX = 20.0
_MN = 100.0
_NL = 3
_BLK = 1024
_PAD_UNIT = 32768

_TW = 128


def _silu(x):
    return x * jax.nn.sigmoid(x)


def _ln(x, g, b):
    m = jnp.mean(x, -1, keepdims=True)
    v = jnp.mean((x - m) ** 2, -1, keepdims=True)
    return (x - m) / jnp.sqrt(v + 1e-5) * g + b


def _mlp(x, p, residual=None):
    h = _silu(x @ p['W1'] + p['b1'])
    h = _silu(h @ p['W2'] + p['b2'])
    if residual is not None:
        h = h + residual
    return _ln(h, p['g'], p['b'])


def _norm(x, axis=-1, keepdims=False):
    return jnp.sqrt(jnp.clip(jnp.sum(x * x, axis=axis, keepdims=keepdims), 1e-8, None))


def _gvp(s, V, p):
    Vh = jnp.einsum('nic,ih->nhc', V, p['Wh'])
    vn = _norm(Vh, axis=-1)
    so = _silu(jnp.concatenate([s, vn], -1) @ p['Ws'] + p['bs'])
    Vo = jnp.einsum('nhc,ho->noc', Vh, p['Wv'])
    gate = jax.nn.sigmoid(so @ p['Wg'] + p['bg'])
    return so, Vo * gate[..., None]


def _structured_mats(Wh, Wv):
    I3 = jnp.eye(3, dtype=jnp.float32)
    wh_last = Wh[16]
    WhK = (Wh[:16][:, None, :, None] * I3[None, :, None, :]).reshape(48, 51)
    M3 = (wh_last[None, :, None] * I3[:, None, :]).reshape(3, 51)
    K = jnp.repeat(jnp.eye(17, dtype=jnp.float32), 3, axis=0)
    WK = (Wv[:, None, :, None] * I3[None, :, None, :]).reshape(51, 48)
    RG = jnp.repeat(jnp.eye(16, dtype=jnp.float32), 3, axis=1)
    return WhK, M3, K, WK, RG


def _msg_body(mode, *refs):
    if mode == 0:
        (ef, gs, gd, We1, be1, We2, be2, ge, be,
         Wsd, Wz, bs, WhK, M3, K, WK, RG, Wg, bg, msg_o, e_o) = refs
    elif mode == 1:
        (e_r, gs, gd, Wsd, Wz, bs, WhK, M3, K, WK, RG, Wg, bg, msg_o) = refs
    else:
        (e_r, gs, gd, Wsd, Wz, bs, WhK, M3, K, WK, RG, Wg, bg,
         W1sd, W1z, b1, W2, b2, lg, lb, msg_o) = refs

    gsv = gs[:]
    gdv = gd[:]
    ssd = jnp.concatenate([gsv[:, 0:64], gdv[:, 0:64]], 1)
    ps = gsv[:, 64:67]
    pd = gdv[:, 64:67]
    Vg = gsv[:, 67:115]

    diff = ps - pd
    n2 = jnp.sum(diff * diff, axis=1, keepdims=True)
    dij = jnp.sqrt(jnp.clip(n2, 1e-8, None)) + 1e-8
    xd = diff / dij
    sigma = _DMAX / _R
    mu = lax.broadcasted_iota(jnp.int32, (1, _R), 1).astype(jnp.float32) * (
        _DMAX / (_R - 1))
    dmat = jnp.exp(-(((dij - mu) / sigma) ** 2))

    if mode == 0:
        h = _silu(ef[:] @ We1[:] + be1[:])
        h = _silu(h @ We2[:] + be2[:])
        ev = _ln(h, ge[:], be[:])
        e_o[:] = ev
    else:
        ev = e_r[:]

    if mode == 2:
        h1 = _silu(ssd @ W1sd[:] +
                   jnp.concatenate([ev, dmat], 1) @ W1z[:] + b1[:])
        h2 = _silu(h1 @ W2[:] + b2[:])
        ev = _ln(h2 + ev, lg[:], lb[:])

    Vh = Vg @ WhK[:] + xd @ M3[:]
    vn = jnp.sqrt(jnp.clip((Vh * Vh) @ K[:], 1e-8, None))
    z = ssd @ Wsd[:] + jnp.concatenate([ev, dmat, vn], 1) @ Wz[:] + bs[:]
    so = _silu(z)
    gate = jax.nn.sigmoid(so @ Wg[:] + bg[:])
    outV = (Vh @ WK[:]) * (gate @ RG[:])
    msg_o[:] = jnp.concatenate([so, outV], 1)


def _msg_layer(mode, Ep, data_arrays, weights):
    grid = Ep // _BLK
    in_specs = [pl.BlockSpec((_BLK, a.shape[1]), lambda i: (i, 0))
                for a in data_arrays]
    in_specs += [pl.BlockSpec(w.shape, lambda i: (0, 0)) for w in weights]
    out_shapes = [jax.ShapeDtypeStruct((Ep, 112), jnp.float32)]
    out_specs = [pl.BlockSpec((_BLK, 112), lambda i: (i, 0))]
    if mode == 0:
        out_shapes.append(jax.ShapeDtypeStruct((Ep, _EF), jnp.float32))
        out_specs.append(pl.BlockSpec((_BLK, _EF), lambda i: (i, 0)))
    fn = pl.pallas_call(
        functools.partial(_msg_body, mode),
        grid=(grid,),
        in_specs=in_specs,
        out_specs=out_specs,
        out_shape=out_shapes,
        compiler_params=pltpu.CompilerParams(
            dimension_semantics=("arbitrary",)),
    )
    return fn(*data_arrays, *weights)


def _msg_weights(mode, params, u_mlp=None):
    p = params['msg']
    Ws, bs, Wg, bg = p['Ws'], p['bs'], p['Wg'], p['bg']
    WhK, M3, K, WK, RG = _structured_mats(p['Wh'], p['Wv'])
    r = lambda x: x.reshape(1, -1)
    Wsd = Ws[0:128]
    Wz = jnp.concatenate([Ws[128:160], Ws[160:176], Ws[176:193]], 0)
    w = [Wsd, Wz, r(bs), WhK, M3, K, WK, RG, Wg, r(bg)]
    if mode == 2:
        W1 = u_mlp['W1']
        W1sd = W1[0:128]
        W1z = jnp.concatenate([W1[128:160], W1[160:176]], 0)
        w += [W1sd, W1z, r(u_mlp['b1']), u_mlp['W2'], r(u_mlp['b2']),
              r(u_mlp['g']), r(u_mlp['b'])]
    return w


_NSC = 2
_NTILE = 16
_NW = _NSC * _NTILE
_CHUNK = 128
_NBUF = 4


def _sc_gather_body(per_w, D, table, idx, out, idx_v, bufs, gsem, osem):
    wid = lax.axis_index("s") * _NSC + lax.axis_index("c")
    base = wid * per_w
    pltpu.sync_copy(idx.at[pl.ds(base, per_w)], idx_v)
    steps = per_w // _NBUF

    def it(jj, carry):
        row0 = (base + jj * _NBUF) * _CHUNK

        @pl.when(jj > 0)
        def _():
            for t in range(_NBUF):
                pltpu.make_async_copy(out.at[pl.ds(0, _CHUNK)], bufs[t],
                                      osem).wait()

        cps = [pltpu.async_copy(table.at[idx_v.at[jj * _NBUF + t]], bufs[t],
                                gsem) for t in range(_NBUF)]
        for t in range(_NBUF):
            cps[t].wait()
        for t in range(_NBUF):
            pltpu.async_copy(bufs[t], out.at[pl.ds(row0 + t * _CHUNK, _CHUNK)],
                             osem)
        return carry

    lax.fori_loop(0, steps, it, 0)
    for t in range(_NBUF):
        pltpu.make_async_copy(out.at[pl.ds(0, _CHUNK)], bufs[t], osem).wait()


def _sc_gather(table, idx2d, Ep):
    D = table.shape[1]
    per_w = (Ep // _CHUNK) // _NW
    mesh = plsc.VectorSubcoreMesh(core_axis_name="c", subcore_axis_name="s")
    scratch = [pltpu.VMEM((per_w, _CHUNK), jnp.int32)]
    scratch += [pltpu.VMEM((_CHUNK, D), jnp.float32) for _ in range(_NBUF)]
    scratch += [pltpu.SemaphoreType.DMA, pltpu.SemaphoreType.DMA]

    @functools.partial(
        pl.kernel,
        out_type=jax.ShapeDtypeStruct((Ep, D), jnp.float32),
        mesh=mesh,
        scratch_types=scratch,
    )
    def k(table_r, idx_r, out_r, idx_v, b0, b1, b2, b3, gsem, osem):
        _sc_gather_body(per_w, D, table_r, idx_r, out_r, idx_v,
                        [b0, b1, b2, b3], gsem, osem)

    return k(table, idx2d)


def _pad_cols(x, w):
    return jnp.pad(x, ((0, 0), (0, w - x.shape[1])))


def kernel(atom_types, positions, edge_features, params, edge_index):
    N = atom_types.shape[0]
    E = edge_features.shape[0]
    Ep = ((E + _PAD_UNIT - 1) // _PAD_UNIT) * _PAD_UNIT
    npad = Ep - E

    src = edge_index[0]
    dst = edge_index[1]
    pad_ids = jnp.arange(npad, dtype=src.dtype)
    srcp = jnp.concatenate([src, pad_ids % N])
    dstp = jnp.concatenate([dst, N + (pad_ids % 64)])
    src2d = srcp.reshape(Ep // _CHUNK, _CHUNK)
    dst2d = dstp.reshape(Ep // _CHUNK, _CHUNK)
    efp = jnp.pad(edge_features.astype(jnp.float32), ((0, npad), (0, 3)))

    s = _mlp(atom_types, params['scalar_emb'])
    V = jnp.zeros((N, _VN, 3), jnp.float32)
    pos = positions

    e = None
    for i in range(_NL):
        cp = params['convs'][i]
        mode = i
        T = _pad_cols(jnp.concatenate([s, pos, V.reshape(N, 48)], 1), _TW)
        T = jnp.concatenate([T, jnp.zeros((64, _TW))], 0)

        Gsrc = _sc_gather(T, src2d, Ep)
        Gdst = _sc_gather(T, dst2d, Ep)

        weights = _msg_weights(mode, cp,
                               params['edge_upd'][0] if mode == 2 else None)
        if mode == 0:
            ee = params['edge_emb']
            We1 = jnp.concatenate([ee['W1'], jnp.zeros((3, _EF))], 0)
            r = lambda x: x.reshape(1, -1)
            emb_w = [We1, r(ee['b1']), ee['W2'], r(ee['b2']),
                     r(ee['g']), r(ee['b'])]
            msg, e = _msg_layer(0, Ep, [efp, Gsrc, Gdst], emb_w + weights)
        else:
            (msg,) = _msg_layer(mode, Ep, [e, Gsrc, Gdst], weights)

        agg = jax.ops.segment_sum(msg, dstp, num_segments=N + 64)[:N]
        s = s + agg[:, :64] / _MN
        V = V + agg[:, 64:].reshape(N, _VN, 3) / _MN

        ds, dV = _gvp(s, V, cp['upd'])
        s = _ln(s + ds, cp['ln_g'], cp['ln_b'])
        V = V + dV

        if i == 1:
            pu = params['pos_upd'][0]
            s1, v1 = _gvp(s, V, pu['g1'])
            s2, v2 = _gvp(s1, v1, pu['g2'])
            s3, v3 = _gvp(s2, v2, pu['g3'])
            pos = pos + v3[:, 0, :]

    pooled = jnp.mean(s, axis=0, keepdims=True)
    pr = params['pred']
    return _silu(pooled @ pr['W1'] + pr['b1']) @ pr['W2'] + pr['b2']
